# Initial kernel scaffold; baseline (speedup 1.0000x reference)
import functools

import jax
import jax.numpy as jnp
from jax import lax
from jax.experimental import pallas as pl
from jax.experimental.pallas import tpu as pltpu
from jax.experimental.pallas import tpu_sc as plsc

NC = 2
NS = 16
L = 16
CHUNK = 128


def _round_up(v, m):
    return (v + m - 1) // m * m


def _deg_body(npt, ept, dst_hbm, degp_hbm, idx_v, hist_v, red_v, hist2_v,
              stack_sh):
    c = lax.axis_index("c")
    s = lax.axis_index("s")
    wid = s * NC + c
    n_pad = npt * NS

    def zero_one(k, _):
        hist_v[pl.ds(k * L, L)] = jnp.zeros((L,), jnp.float32)
        return 0
    lax.fori_loop(0, n_pad // L, zero_one, 0)

    pltpu.sync_copy(dst_hbm.at[pl.ds(wid * ept, ept)], idx_v)
    ones = jnp.ones((L,), jnp.float32)

    def hist_one(j, _):
        d16 = idx_v[pl.ds(j * L, L)]
        plsc.addupdate_scatter(hist_v, [d16], ones)
        return 0
    lax.fori_loop(0, ept // L, hist_one, 0)

    pltpu.sync_copy(hist_v, stack_sh.at[s])
    plsc.subcore_barrier()

    for r in range(NS):
        pltpu.sync_copy(stack_sh.at[r, pl.ds(s * npt, npt)], red_v.at[r])

    def red_one(k, _):
        v = red_v[0, pl.ds(k * L, L)]
        for r in range(1, NS):
            v = v + red_v[r, pl.ds(k * L, L)]
        hist2_v[pl.ds(k * L, L)] = v
        return 0
    lax.fori_loop(0, npt // L, red_one, 0)

    pltpu.sync_copy(hist2_v, degp_hbm.at[c, pl.ds(s * npt, npt)])


def _agg_body(npt, chunks, g_hbm, src_hbm, dst_hbm, acc_hbm,
              sidx_v, didx_v, rows_v, zbuf_v, sem, acc_sh):
    c = lax.axis_index("c")
    s = lax.axis_index("s")
    wid = s * NC + c
    d = rows_v.shape[1]

    for r in range(L):
        for u in range(d // L):
            zbuf_v[r, pl.ds(u * L, L)] = jnp.zeros((L,), jnp.float32)

    def zero_one(j, _):
        pltpu.sync_copy(zbuf_v, acc_sh.at[pl.ds(s * npt + j * L, L)])
        return 0
    lax.fori_loop(0, npt // L, zero_one, 0)
    plsc.subcore_barrier()

    def edge_chunk(k, _):
        base = (wid * chunks + k) * CHUNK
        pltpu.sync_copy(src_hbm.at[pl.ds(base, CHUNK)], sidx_v)
        pltpu.sync_copy(dst_hbm.at[pl.ds(base, CHUNK)], didx_v)
        pltpu.async_copy(g_hbm.at[sidx_v], rows_v, sem).wait()
        pltpu.sync_copy(rows_v, acc_sh.at[didx_v], add=True)
        return 0
    lax.fori_loop(0, chunks, edge_chunk, 0)

    plsc.subcore_barrier()
    pltpu.sync_copy(acc_sh.at[pl.ds(s * npt, npt)],
                    acc_hbm.at[c, pl.ds(s * npt, npt)])


def _mm_body(x_ref, w_ref, degp_ref, h_ref, g_ref):
    deg = degp_ref[0, :] + degp_ref[1, :] + 1.0
    dinv = lax.rsqrt(deg)
    h = jnp.dot(x_ref[...], w_ref[...], preferred_element_type=jnp.float32)
    h_ref[...] = h
    g_ref[...] = h * dinv[:, None]


def _fin_body(acc_ref, degp_ref, h_ref, b_ref, a_ref, o_ref):
    deg = degp_ref[0, :] + degp_ref[1, :] + 1.0
    dinv = lax.rsqrt(deg)
    acc = acc_ref[0] + acc_ref[1]
    pre = dinv[:, None] * acc + (dinv * dinv)[:, None] * h_ref[...] \
        + b_ref[...]
    o_ref[...] = jnp.where(pre >= 0.0, pre, a_ref[...] * pre)


def kernel(x, edge_index, W, b, a):
    n, d_in = x.shape
    d_hid = W.shape[1]
    e = edge_index.shape[1]

    npt = _round_up((n + 1 + NS - 1) // NS, L)
    n_pad = npt * NS
    ept = _round_up((e + NC * NS - 1) // (NC * NS), CHUNK)
    e_pad = ept * NC * NS
    chunks = ept // CHUNK

    src = edge_index[0]
    dst = edge_index[1]
    pad = e_pad - e
    src_p = jnp.concatenate([src, jnp.zeros((pad,), jnp.int32)])
    dst_p = jnp.concatenate([dst, jnp.full((pad,), n, jnp.int32)])
    x_p = jnp.pad(x, ((0, n_pad - n), (0, 0)))

    mesh = plsc.VectorSubcoreMesh(core_axis_name="c", subcore_axis_name="s")

    deg_call = pl.kernel(
        functools.partial(_deg_body, npt, ept),
        out_type=jax.ShapeDtypeStruct((NC, n_pad), jnp.float32),
        mesh=mesh,
        scratch_types=[
            pltpu.VMEM((ept,), jnp.int32),
            pltpu.VMEM((n_pad,), jnp.float32),
            pltpu.VMEM((NS, npt), jnp.float32),
            pltpu.VMEM((npt,), jnp.float32),
            pltpu.VMEM_SHARED((NS, n_pad), jnp.float32),
        ],
    )
    degp = deg_call(dst_p)

    nb = n_pad // 1024
    h, g = pl.pallas_call(
        _mm_body,
        grid=(nb,),
        in_specs=[
            pl.BlockSpec((1024, d_in), lambda i: (i, 0)),
            pl.BlockSpec((d_in, d_hid), lambda i: (0, 0)),
            pl.BlockSpec((NC, 1024), lambda i: (0, i)),
        ],
        out_specs=[
            pl.BlockSpec((1024, d_hid), lambda i: (i, 0)),
            pl.BlockSpec((1024, d_hid), lambda i: (i, 0)),
        ],
        out_shape=[
            jax.ShapeDtypeStruct((n_pad, d_hid), jnp.float32),
            jax.ShapeDtypeStruct((n_pad, d_hid), jnp.float32),
        ],
    )(x_p, W, degp)

    agg_call = pl.kernel(
        functools.partial(_agg_body, npt, chunks),
        out_type=jax.ShapeDtypeStruct((NC, n_pad, d_hid), jnp.float32),
        mesh=mesh,
        scratch_types=[
            pltpu.VMEM((CHUNK,), jnp.int32),
            pltpu.VMEM((CHUNK,), jnp.int32),
            pltpu.VMEM((CHUNK, d_hid), jnp.float32),
            pltpu.VMEM((L, d_hid), jnp.float32),
            pltpu.SemaphoreType.DMA,
            pltpu.VMEM_SHARED((n_pad, d_hid), jnp.float32),
        ],
    )
    acc = agg_call(g, src_p, dst_p)

    bn = 1000
    out = pl.pallas_call(
        _fin_body,
        grid=(n // bn,),
        in_specs=[
            pl.BlockSpec((NC, bn, d_hid), lambda i: (0, i, 0)),
            pl.BlockSpec((NC, bn), lambda i: (0, i)),
            pl.BlockSpec((bn, d_hid), lambda i: (i, 0)),
            pl.BlockSpec((1, d_hid), lambda i: (0, 0)),
            pl.BlockSpec((1, d_hid), lambda i: (0, 0)),
        ],
        out_specs=pl.BlockSpec((bn, d_hid), lambda i: (i, 0)),
        out_shape=jax.ShapeDtypeStruct((n, d_hid), jnp.float32),
    )(acc, degp, h, b.reshape(1, -1), a.reshape(1, -1))
    return out


# speedup vs baseline: 16.6785x; 16.6785x over previous
import functools

import jax
import jax.numpy as jnp
from jax import lax
from jax.experimental import pallas as pl
from jax.experimental.pallas import tpu as pltpu
from jax.experimental.pallas import tpu_sc as plsc

NC = 2
NS = 16
L = 16
CHUNK = 128


def _round_up(v, m):
    return (v + m - 1) // m * m


def _deg_body(npt, ept, dst_hbm, degp_hbm, idx_v, hist_v, red_v, hist2_v,
              stack_sh):
    c = lax.axis_index("c")
    s = lax.axis_index("s")
    wid = s * NC + c
    n_pad = npt * NS

    def zero_one(k, _):
        hist_v[pl.ds(k * L, L)] = jnp.zeros((L,), jnp.float32)
        return 0
    lax.fori_loop(0, n_pad // L, zero_one, 0)

    pltpu.sync_copy(dst_hbm.at[pl.ds(wid * ept, ept)], idx_v)
    ones = jnp.ones((L,), jnp.float32)

    def hist_one(j, _):
        d16 = idx_v[pl.ds(j * L, L)]
        plsc.addupdate_scatter(hist_v, [d16], ones)
        return 0
    lax.fori_loop(0, ept // L, hist_one, 0)

    pltpu.sync_copy(hist_v, stack_sh.at[s])
    plsc.subcore_barrier()

    for r in range(NS):
        pltpu.sync_copy(stack_sh.at[r, pl.ds(s * npt, npt)], red_v.at[r])

    def red_one(k, _):
        v = red_v[0, pl.ds(k * L, L)]
        for r in range(1, NS):
            v = v + red_v[r, pl.ds(k * L, L)]
        hist2_v[pl.ds(k * L, L)] = v
        return 0
    lax.fori_loop(0, npt // L, red_one, 0)

    pltpu.sync_copy(hist2_v, degp_hbm.at[c, pl.ds(s * npt, npt)])


def _agg_body(npt, chunks, g_hbm, src_hbm, dst_hbm, acc_hbm,
              sidx_v, didx_v, rows_v, zbuf_v, sem, acc_sh):
    c = lax.axis_index("c")
    s = lax.axis_index("s")
    wid = s * NC + c
    d = rows_v.shape[1]

    for r in range(L):
        for u in range(d // L):
            zbuf_v[r, pl.ds(u * L, L)] = jnp.zeros((L,), jnp.float32)

    def zero_one(j, _):
        pltpu.sync_copy(zbuf_v, acc_sh.at[pl.ds(s * npt + j * L, L)])
        return 0
    lax.fori_loop(0, npt // L, zero_one, 0)
    plsc.subcore_barrier()

    def edge_chunk(k, _):
        base = (wid * chunks + k) * CHUNK
        pltpu.sync_copy(src_hbm.at[pl.ds(base, CHUNK)], sidx_v)
        pltpu.sync_copy(dst_hbm.at[pl.ds(base, CHUNK)], didx_v)
        pltpu.async_copy(g_hbm.at[sidx_v], rows_v, sem).wait()
        pltpu.sync_copy(rows_v, acc_sh.at[didx_v], add=True)
        return 0
    lax.fori_loop(0, chunks, edge_chunk, 0)

    plsc.subcore_barrier()
    pltpu.sync_copy(acc_sh.at[pl.ds(s * npt, npt)],
                    acc_hbm.at[c, pl.ds(s * npt, npt)])


def _mm_body(x_ref, w_ref, degp_ref, h_ref, g_ref):
    deg = degp_ref[0, :] + degp_ref[1, :] + 1.0
    dinv = lax.rsqrt(deg)
    h = jnp.dot(x_ref[...], w_ref[...], preferred_element_type=jnp.float32)
    h_ref[...] = h
    g_ref[...] = h * dinv[:, None]


def _fin_body(acc_ref, degp_ref, h_ref, b_ref, a_ref, o_ref):
    deg = degp_ref[0, :] + degp_ref[1, :] + 1.0
    dinv = lax.rsqrt(deg)
    acc = acc_ref[0] + acc_ref[1]
    pre = dinv[:, None] * acc + (dinv * dinv)[:, None] * h_ref[...] \
        + b_ref[...]
    o_ref[...] = jnp.where(pre >= 0.0, pre, a_ref[...] * pre)


def kernel(x, edge_index, W, b, a):
    n, d_in = x.shape
    d_hid = W.shape[1]
    e = edge_index.shape[1]

    npt = _round_up((n + 1 + NS - 1) // NS, L)
    n_pad = npt * NS
    ept = _round_up((e + NC * NS - 1) // (NC * NS), CHUNK)
    e_pad = ept * NC * NS
    chunks = ept // CHUNK

    src = edge_index[0]
    dst = edge_index[1]
    pad = e_pad - e
    src_p = jnp.concatenate([src, jnp.zeros((pad,), jnp.int32)])
    dst_p = jnp.concatenate([dst, jnp.full((pad,), n, jnp.int32)])
    x_p = jnp.pad(x, ((0, n_pad - n), (0, 0)))

    mesh = plsc.VectorSubcoreMesh(core_axis_name="c", subcore_axis_name="s")

    deg_call = pl.kernel(
        functools.partial(_deg_body, npt, ept),
        out_type=jax.ShapeDtypeStruct((NC, n_pad), jnp.float32),
        mesh=mesh,
        scratch_types=[
            pltpu.VMEM((ept,), jnp.int32),
            pltpu.VMEM((n_pad,), jnp.float32),
            pltpu.VMEM((NS, npt), jnp.float32),
            pltpu.VMEM((npt,), jnp.float32),
            pltpu.VMEM_SHARED((NS, n_pad), jnp.float32),
        ],
        compiler_params=pltpu.CompilerParams(needs_layout_passes=False),
    )
    degp = deg_call(dst_p)

    nb = n_pad // 1024
    h, g = pl.pallas_call(
        _mm_body,
        grid=(nb,),
        in_specs=[
            pl.BlockSpec((1024, d_in), lambda i: (i, 0)),
            pl.BlockSpec((d_in, d_hid), lambda i: (0, 0)),
            pl.BlockSpec((NC, 1024), lambda i: (0, i)),
        ],
        out_specs=[
            pl.BlockSpec((1024, d_hid), lambda i: (i, 0)),
            pl.BlockSpec((1024, d_hid), lambda i: (i, 0)),
        ],
        out_shape=[
            jax.ShapeDtypeStruct((n_pad, d_hid), jnp.float32),
            jax.ShapeDtypeStruct((n_pad, d_hid), jnp.float32),
        ],
    )(x_p, W, degp)

    agg_call = pl.kernel(
        functools.partial(_agg_body, npt, chunks),
        out_type=jax.ShapeDtypeStruct((NC, n_pad, d_hid), jnp.float32),
        mesh=mesh,
        scratch_types=[
            pltpu.VMEM((CHUNK,), jnp.int32),
            pltpu.VMEM((CHUNK,), jnp.int32),
            pltpu.VMEM((CHUNK, d_hid), jnp.float32),
            pltpu.VMEM((L, d_hid), jnp.float32),
            pltpu.SemaphoreType.DMA,
            pltpu.VMEM_SHARED((n_pad, d_hid), jnp.float32),
        ],
    )
    acc = agg_call(g, src_p, dst_p)

    bn = 1024
    out = pl.pallas_call(
        _fin_body,
        grid=(n_pad // bn,),
        in_specs=[
            pl.BlockSpec((NC, bn, d_hid), lambda i: (0, i, 0)),
            pl.BlockSpec((NC, bn), lambda i: (0, i)),
            pl.BlockSpec((bn, d_hid), lambda i: (i, 0)),
            pl.BlockSpec((1, d_hid), lambda i: (0, 0)),
            pl.BlockSpec((1, d_hid), lambda i: (0, 0)),
        ],
        out_specs=pl.BlockSpec((bn, d_hid), lambda i: (i, 0)),
        out_shape=jax.ShapeDtypeStruct((n_pad, d_hid), jnp.float32),
    )(acc, degp, h, b.reshape(1, -1), a.reshape(1, -1))
    return out[:n]
